# baseline (speedup 1.0000x reference)
import functools

import jax
import jax.numpy as jnp
from jax import lax
from jax.experimental import pallas as pl
from jax.experimental.pallas import tpu as pltpu
from jax.experimental.pallas import tpu_sc as plsc

_NC = 2
_NS = 16
_NW = _NC * _NS

_CHUNK = 128


def _gather_rows(table, idx2d):
    n_chunks, _ = idx2d.shape
    _, d = table.shape
    cpw = n_chunks // _NW
    rows_per_worker = cpw * _CHUNK

    mesh = plsc.VectorSubcoreMesh(core_axis_name="c", subcore_axis_name="s")

    @functools.partial(
        pl.kernel,
        mesh=mesh,
        compiler_params=pltpu.CompilerParams(use_tc_tiling_on_sc=False),
        out_type=jax.ShapeDtypeStruct((n_chunks * _CHUNK, d), jnp.float32),
        scratch_types=[
            pltpu.VMEM((cpw, _CHUNK), jnp.int32),
            pltpu.VMEM((_CHUNK, d), jnp.float32),
            pltpu.SemaphoreType.DMA,
        ],
    )
    def k(table_hbm, idx_hbm, out_hbm, idx_v, rows_v, gsem):
        wid = lax.axis_index("s") * _NC + lax.axis_index("c")
        chunk_base = wid * cpw
        row_base = wid * rows_per_worker
        pltpu.sync_copy(idx_hbm.at[pl.ds(chunk_base, cpw)], idx_v)

        def body(j, _):
            pltpu.async_copy(table_hbm.at[idx_v.at[j]], rows_v, gsem).wait()
            pltpu.sync_copy(rows_v, out_hbm.at[pl.ds(row_base + j * _CHUNK, _CHUNK)])
            return 0

        lax.fori_loop(0, cpw, body, 0)

    return k(table, idx2d)


def kernel(X_word, X_char, word_embed):
    batch, seq = X_word.shape
    d = word_embed.shape[1]
    idx = X_word.reshape(-1).astype(jnp.int32)
    n = idx.shape[0]
    idx2d = idx.reshape(n // _CHUNK, _CHUNK)
    out = _gather_rows(word_embed, idx2d)
    return out.reshape(batch, seq, d)


# speedup vs baseline: 1.1151x; 1.1151x over previous
import functools

import jax
import jax.numpy as jnp
from jax import lax
from jax.experimental import pallas as pl
from jax.experimental.pallas import tpu as pltpu
from jax.experimental.pallas import tpu_sc as plsc

_NC = 2
_NS = 16
_NW = _NC * _NS

_CHUNK = 128


_K = 4
_BURST = _K * _CHUNK


def _gather_rows(table, idx2d):
    n_chunks, _ = idx2d.shape
    _, d = table.shape
    cpw = n_chunks // _NW
    rows_per_worker = cpw * _CHUNK
    n_bursts = cpw // _K

    mesh = plsc.VectorSubcoreMesh(core_axis_name="c", subcore_axis_name="s")

    @functools.partial(
        pl.kernel,
        mesh=mesh,
        compiler_params=pltpu.CompilerParams(use_tc_tiling_on_sc=False),
        out_type=jax.ShapeDtypeStruct((n_chunks * _CHUNK, d), jnp.float32),
        scratch_types=[
            pltpu.VMEM((cpw, _CHUNK), jnp.int32),
            pltpu.VMEM((2, _BURST, d), jnp.float32),
            pltpu.SemaphoreType.DMA,
            pltpu.SemaphoreType.DMA,
            pltpu.SemaphoreType.DMA,
            pltpu.SemaphoreType.DMA,
        ],
    )
    def k(table_hbm, idx_hbm, out_hbm, idx_v, rows_v, g0, g1, o0, o1):
        wid = lax.axis_index("s") * _NC + lax.axis_index("c")
        chunk_base = wid * cpw
        row_base = wid * rows_per_worker
        gsems = (g0, g1)
        osems = (o0, o1)
        pltpu.sync_copy(idx_hbm.at[pl.ds(chunk_base, cpw)], idx_v)

        def fire(t, b):
            for kk in range(_K):
                pltpu.async_copy(
                    table_hbm.at[idx_v.at[t * _K + kk]],
                    rows_v.at[b, pl.ds(kk * _CHUNK, _CHUNK)],
                    gsems[b],
                )

        fire(0, 0)
        fire(1, 1)

        def body(g, _):
            for b in range(2):
                t = g * 2 + b
                pltpu.make_async_copy(
                    out_hbm.at[pl.ds(row_base, _BURST)], rows_v.at[b], gsems[b]
                ).wait()
                w = pltpu.async_copy(
                    rows_v.at[b],
                    out_hbm.at[pl.ds(row_base + t * _BURST, _BURST)],
                    osems[b],
                )
                w.wait()

                @pl.when(t + 2 < n_bursts)
                def _():
                    fire(t + 2, b)

            return 0

        lax.fori_loop(0, n_bursts // 2, body, 0)

    return k(table, idx2d)


def kernel(X_word, X_char, word_embed):
    batch, seq = X_word.shape
    d = word_embed.shape[1]
    idx = X_word.reshape(-1).astype(jnp.int32)
    n = idx.shape[0]
    idx2d = idx.reshape(n // _CHUNK, _CHUNK)
    out = _gather_rows(word_embed, idx2d)
    return out.reshape(batch, seq, d)


# speedup vs baseline: 1.1181x; 1.0027x over previous
import functools

import jax
import jax.numpy as jnp
from jax import lax
from jax.experimental import pallas as pl
from jax.experimental.pallas import tpu as pltpu
from jax.experimental.pallas import tpu_sc as plsc

_NC = 2
_NS = 16
_NW = _NC * _NS

_QB = 2


def _lookup(table, xw):
    b, s = xw.shape
    _, d = table.shape
    rpw = b // _NW
    n_bursts = rpw // _QB
    s0 = (s // 2 + 127) // 128 * 128
    s1 = s - s0

    mesh = plsc.VectorSubcoreMesh(core_axis_name="c", subcore_axis_name="s")

    @functools.partial(
        pl.kernel,
        mesh=mesh,
        compiler_params=pltpu.CompilerParams(use_tc_tiling_on_sc=False),
        out_type=jax.ShapeDtypeStruct((b, s, d), jnp.float32),
        scratch_types=[
            pltpu.VMEM((rpw, s), jnp.int32),
            pltpu.VMEM((2, _QB, s, d), jnp.float32),
            pltpu.SemaphoreType.DMA,
            pltpu.SemaphoreType.DMA,
            pltpu.SemaphoreType.DMA,
            pltpu.SemaphoreType.DMA,
        ],
    )
    def k(table_hbm, xw_hbm, out_hbm, idx_v, rows_v, g0, g1, o0, o1):
        wid = lax.axis_index("s") * _NC + lax.axis_index("c")
        row_base = wid * rpw
        gsems = (g0, g1)
        osems = (o0, o1)
        pltpu.sync_copy(xw_hbm.at[pl.ds(row_base, rpw)], idx_v)

        def fire(t, buf):
            for q in range(_QB):
                r = t * _QB + q
                pltpu.async_copy(
                    table_hbm.at[idx_v.at[r, pl.ds(0, s0)]],
                    rows_v.at[buf, q, pl.ds(0, s0)],
                    gsems[buf],
                )
                pltpu.async_copy(
                    table_hbm.at[idx_v.at[r, pl.ds(s0, s1)]],
                    rows_v.at[buf, q, pl.ds(s0, s1)],
                    gsems[buf],
                )

        fire(0, 0)
        fire(1, 1)

        def body(g, _):
            for buf in range(2):
                t = g * 2 + buf
                pltpu.make_async_copy(
                    out_hbm.at[pl.ds(row_base, _QB)], rows_v.at[buf], gsems[buf]
                ).wait()
                w = pltpu.async_copy(
                    rows_v.at[buf],
                    out_hbm.at[pl.ds(row_base + t * _QB, _QB)],
                    osems[buf],
                )
                w.wait()

                @pl.when(t + 2 < n_bursts)
                def _():
                    fire(t + 2, buf)

            return 0

        lax.fori_loop(0, n_bursts // 2, body, 0)

    return k(table, xw)


def kernel(X_word, X_char, word_embed):
    return _lookup(word_embed, X_word.astype(jnp.int32))
